# baseline (speedup 1.0000x reference)
import functools

import jax
import jax.numpy as jnp
from jax import lax
from jax.experimental import pallas as pl
from jax.experimental.pallas import tpu as pltpu
from jax.experimental.pallas import tpu_sc as plsc

_NEG_INF = float("-inf")
_BIG = 1 << 20
_NC = 2
_NS = 16
_RB = 256


def _router_body(x_ref, wr_ref, br_ref, wn_ref, bn_ref, nz_ref,
                 pos_ref, gate_ref, tok_ref, gslot_ref, meta_ref,
                 counts_ref, *, cap):
    x = x_ref[...]
    E = wr_ref.shape[1]
    F = x.shape[0]
    dn = (((0,), (1,)), ((), ()))
    logits = lax.dot_general(wr_ref[...], x, dn,
                             preferred_element_type=jnp.float32)
    logits = logits + br_ref[...]
    nlogits = lax.dot_general(wn_ref[...], x, dn,
                              preferred_element_type=jnp.float32)
    nlogits = nlogits + bn_ref[...]
    sp = jnp.log1p(jnp.exp(-jnp.abs(nlogits))) + jnp.maximum(nlogits, 0.0)
    noisy = logits + nz_ref[...] * sp

    e_iota = lax.broadcasted_iota(jnp.int32, noisy.shape, 0)
    top1v = jnp.max(noisy, axis=0, keepdims=True)
    top1i = jnp.min(jnp.where(noisy == top1v, e_iota, E), axis=0,
                    keepdims=True)
    masked = jnp.where(e_iota == top1i, _NEG_INF, noisy)
    top2v = jnp.max(masked, axis=0, keepdims=True)
    top2i = jnp.min(jnp.where(masked == top2v, e_iota, E), axis=0,
                    keepdims=True)

    ed = jnp.exp(top2v - top1v)
    denom = 1.0 + ed
    g1 = 1.0 / denom
    g2 = ed / denom

    sel1 = e_iota == top1i
    sel2 = e_iota == top2i
    sel = jnp.logical_or(sel1, sel2)
    self32 = sel.astype(jnp.float32)

    r_iota = lax.broadcasted_iota(jnp.int32, (F, F), 0)
    c_iota = lax.broadcasted_iota(jnp.int32, (F, F), 1)
    ut = (r_iota <= c_iota).astype(jnp.float32)
    ranks = jnp.dot(self32, ut, preferred_element_type=jnp.float32)
    posf = jnp.where(sel, ranks - 1.0, float(_BIG))
    pos = posf.astype(jnp.int32)

    gate = jnp.where(sel1, g1, jnp.where(sel2, g2, 0.0))
    gate = jnp.where(pos < cap, gate, 0.0)

    pos_ref[...] = pos
    gate_ref[...] = gate

    counts = ranks[:, F - 1:F]
    e_col = lax.broadcasted_iota(jnp.int32, (E, 1), 0)
    estar = jnp.min(jnp.where(counts < cap, e_col, E), axis=0,
                    keepdims=True)
    counts_ref[...] = counts.astype(jnp.int32)

    o_drop = estar * cap + (cap - 1)
    pos1 = jnp.sum(jnp.where(sel1, posf, 0.0), axis=0, keepdims=True)
    pos2 = jnp.sum(jnp.where(sel2, posf, 0.0), axis=0, keepdims=True)
    o1 = jnp.where(pos1 < cap, top1i * cap + pos1.astype(jnp.int32), o_drop)
    o2 = jnp.where(pos2 < cap, top2i * cap + pos2.astype(jnp.int32), o_drop)
    meta_ref[0:1, :] = o1
    meta_ref[1:2, :] = o2

    tok_iota = lax.broadcasted_iota(jnp.int32, (1, F), 1).astype(jnp.float32)
    slot_iota = lax.broadcasted_iota(jnp.int32, (cap, F), 0)
    for e in range(E):
        p_e = (pos[e:e + 1, :] == slot_iota).astype(jnp.float32)
        tok_col = jnp.sum(p_e * tok_iota, axis=1, keepdims=True)
        g_col = jnp.sum(p_e * gate[e:e + 1, :], axis=1, keepdims=True)
        tok_ref[pl.ds(e * cap, cap), :] = tok_col.astype(jnp.int32)
        gslot_ref[pl.ds(e * cap, cap), :] = g_col


def _ffn_body(counts_ref, ybuf_in_ref, gslot_ref, xg_ref, w1_ref, b1_ref,
              w2_ref, b2_ref, out_ref, y_scr, *, cap, n_ffb, e):
    del ybuf_in_ref
    f = pl.program_id(0)
    count = jnp.minimum(counts_ref[e], cap)
    blocks = []
    start = 0
    while start < cap:
        size = min(_RB, cap - start)
        blocks.append((start, size))
        start += size

    for start, size in blocks:
        @pl.when(start < count)
        def _compute(start=start, size=size):
            rows = pl.ds(start, size)
            h = jnp.dot(xg_ref[rows, :], w1_ref[0],
                        preferred_element_type=jnp.float32)
            h = h + b1_ref[0]
            h = h * 0.5 * (1.0 + lax.erf(h * 0.7071067811865476))
            yb = jnp.dot(h, w2_ref[0], preferred_element_type=jnp.float32)

            @pl.when(f == 0)
            def _init():
                y_scr[rows, :] = yb

            @pl.when(f != 0)
            def _acc():
                y_scr[rows, :] = y_scr[rows, :] + yb

    @pl.when(f == n_ffb - 1)
    def _emit():
        for start, size in blocks:
            rows = pl.ds(start, size)

            @pl.when(start < count)
            def _write(rows=rows):
                out_ref[rows, :] = ((y_scr[rows, :] + b2_ref[0])
                                    * gslot_ref[rows, :])

            @pl.when(start >= count)
            def _zero(rows=rows):
                out_ref[rows, :] = jnp.zeros_like(out_ref[rows, :])


def _gather_body(flat_hbm, tok_hbm, out_hbm, idx_v, rows_v0, rows_v1,
                 sem0, sem1, *, rows_per_w, chunk):
    wid = lax.axis_index("s") * _NC + lax.axis_index("c")
    base = wid * rows_per_w
    pltpu.sync_copy(tok_hbm.at[pl.ds(base, rows_per_w)], idx_v)
    bufs = (rows_v0, rows_v1)
    sems = (sem0, sem1)
    n = rows_per_w // chunk
    cps = [None] * n
    cps[0] = pltpu.async_copy(
        flat_hbm.at[idx_v.at[pl.ds(0, chunk)]], bufs[0], sems[0])
    for c in range(n):
        cps[c].wait()
        if c + 1 < n:
            cps[c + 1] = pltpu.async_copy(
                flat_hbm.at[idx_v.at[pl.ds((c + 1) * chunk, chunk)]],
                bufs[(c + 1) % 2], sems[(c + 1) % 2])
        pltpu.sync_copy(bufs[c % 2],
                        out_hbm.at[pl.ds(base + c * chunk, chunk)])


def _combine_body(ybuf_hbm, o1_hbm, o2_hbm, out_hbm,
                  idx1_v, idx2_v, r1_v, r2_v, sem1, sem2,
                  *, toks_per_w, chunk, d):
    wid = lax.axis_index("s") * _NC + lax.axis_index("c")
    base = wid * toks_per_w
    pltpu.sync_copy(o1_hbm.at[pl.ds(base, toks_per_w)], idx1_v)
    pltpu.sync_copy(o2_hbm.at[pl.ds(base, toks_per_w)], idx2_v)
    n_chunks = toks_per_w // chunk
    n_lane = d // 16
    for c in range(n_chunks):
        cp1 = pltpu.async_copy(
            ybuf_hbm.at[idx1_v.at[pl.ds(c * chunk, chunk)]], r1_v, sem1)
        cp2 = pltpu.async_copy(
            ybuf_hbm.at[idx2_v.at[pl.ds(c * chunk, chunk)]], r2_v, sem2)
        cp1.wait()
        cp2.wait()

        def _row(r, _):
            def _col(j, _):
                for u in range(4):
                    cols = pl.ds(j * 64 + u * 16, 16)
                    r1_v[r, cols] = r1_v[r, cols] + r2_v[r, cols]
                return _
            return lax.fori_loop(0, n_lane // 4, _col, _)

        lax.fori_loop(0, chunk, _row, None)
        pltpu.sync_copy(r1_v, out_hbm.at[pl.ds(base + c * chunk, chunk)])


def kernel(x_BLD, W_route, b_route, W_noise, b_noise, W1, b1, W2, b2):
    Bs, Ls, Ds = x_BLD.shape
    F = Bs * Ls
    E = W_route.shape[1]
    FF = W1.shape[2]
    cap = int(F * 2 / E * 1.25)
    flat = x_BLD.reshape(F, Ds)
    noise_mat = jax.random.normal(jax.random.key(1234), (F, E),
                                  dtype=jnp.float32)

    pos, gate, tok, gslot, meta, counts = pl.pallas_call(
        functools.partial(_router_body, cap=cap),
        out_shape=(
            jax.ShapeDtypeStruct((E, F), jnp.int32),
            jax.ShapeDtypeStruct((E, F), jnp.float32),
            jax.ShapeDtypeStruct((E * cap, 1), jnp.int32),
            jax.ShapeDtypeStruct((E * cap, 1), jnp.float32),
            jax.ShapeDtypeStruct((2, F), jnp.int32),
            jax.ShapeDtypeStruct((E, 1), jnp.int32),
        ),
    )(flat, W_route, b_route.reshape(E, 1), W_noise, b_noise.reshape(E, 1),
      noise_mat.T)

    nw = _NC * _NS
    gpe = 2
    rows_per_w = (gpe * cap) // nw
    mesh = plsc.VectorSubcoreMesh(core_axis_name="c", subcore_axis_name="s",
                                  num_cores=_NC, num_subcores=_NS)
    tok_flat = tok.reshape(E * cap)
    counts1d = counts.reshape(E)
    b1r = b1.reshape(E, 1, FF)
    b2r = b2.reshape(E, 1, Ds)
    n_ffb = 2
    ffb = FF // n_ffb

    sc_gather = pl.kernel(
        functools.partial(_gather_body, rows_per_w=rows_per_w,
                          chunk=rows_per_w),
        out_type=jax.ShapeDtypeStruct((gpe * cap, Ds), jnp.float32),
        mesh=mesh,
        scratch_types=[
            pltpu.VMEM((rows_per_w,), jnp.int32),
            pltpu.VMEM((rows_per_w, Ds), jnp.float32),
            pltpu.VMEM((rows_per_w, Ds), jnp.float32),
            pltpu.SemaphoreType.DMA,
            pltpu.SemaphoreType.DMA,
        ],
    )

    xg_list = [sc_gather(flat, lax.slice(tok_flat, (g * gpe * cap,),
                                         ((g + 1) * gpe * cap,)))
               for g in range(E // gpe)]

    ybuf = jnp.zeros((E * cap, Ds), jnp.float32)
    for e in range(E):
        ybuf = pl.pallas_call(
            functools.partial(_ffn_body, cap=cap, n_ffb=n_ffb, e=e),
            grid=(n_ffb,),
            in_specs=[
                pl.BlockSpec(memory_space=pltpu.SMEM),
                pl.BlockSpec(memory_space=pl.ANY),
                pl.BlockSpec((cap, 1), lambda f, e=e: (e, 0)),
                pl.BlockSpec((cap, Ds), lambda f, e=e: (e % gpe, 0)),
                pl.BlockSpec((1, Ds, ffb), lambda f, e=e: (e, 0, f)),
                pl.BlockSpec((1, 1, ffb), lambda f, e=e: (e, 0, f)),
                pl.BlockSpec((1, ffb, Ds), lambda f, e=e: (e, f, 0)),
                pl.BlockSpec((1, 1, Ds), lambda f, e=e: (e, 0, 0)),
            ],
            out_specs=pl.BlockSpec((cap, Ds), lambda f, e=e: (e, 0)),
            out_shape=jax.ShapeDtypeStruct((E * cap, Ds), jnp.float32),
            scratch_shapes=[
                pltpu.VMEM((cap, Ds), jnp.float32),
            ],
            input_output_aliases={1: 0},
        )(counts1d, ybuf, gslot, xg_list[e // gpe], W1, b1r, W2, b2r)

    toks_per_w = F // nw
    c_chunk = 32
    out = pl.kernel(
        functools.partial(_combine_body, toks_per_w=toks_per_w,
                          chunk=c_chunk, d=Ds),
        out_type=jax.ShapeDtypeStruct((F, Ds), jnp.float32),
        mesh=mesh,
        scratch_types=[
            pltpu.VMEM((toks_per_w,), jnp.int32),
            pltpu.VMEM((toks_per_w,), jnp.int32),
            pltpu.VMEM((c_chunk, Ds), jnp.float32),
            pltpu.VMEM((c_chunk, Ds), jnp.float32),
            pltpu.SemaphoreType.DMA,
            pltpu.SemaphoreType.DMA,
        ],
    )(ybuf, meta[0], meta[1])

    return out.reshape(Bs, Ls, Ds)


# speedup vs baseline: 1.2120x; 1.2120x over previous
import functools

import jax
import jax.numpy as jnp
from jax import lax
from jax.experimental import pallas as pl
from jax.experimental.pallas import tpu as pltpu
from jax.experimental.pallas import tpu_sc as plsc

_NEG_INF = float("-inf")
_BIG = 1 << 20
_NC = 2
_NS = 16
_RB = 256


def _router_body(x_ref, wr_ref, br_ref, wn_ref, bn_ref, nz_ref,
                 pos_ref, gate_ref, tok_ref, gslot_ref, meta_ref,
                 counts_ref, *, cap):
    x = x_ref[...]
    E = wr_ref.shape[1]
    F = x.shape[0]
    dn = (((0,), (1,)), ((), ()))
    logits = lax.dot_general(wr_ref[...], x, dn,
                             preferred_element_type=jnp.float32)
    logits = logits + br_ref[...]
    nlogits = lax.dot_general(wn_ref[...], x, dn,
                              preferred_element_type=jnp.float32)
    nlogits = nlogits + bn_ref[...]
    sp = jnp.log1p(jnp.exp(-jnp.abs(nlogits))) + jnp.maximum(nlogits, 0.0)
    noisy = logits + nz_ref[...] * sp

    e_iota = lax.broadcasted_iota(jnp.int32, noisy.shape, 0)
    top1v = jnp.max(noisy, axis=0, keepdims=True)
    top1i = jnp.min(jnp.where(noisy == top1v, e_iota, E), axis=0,
                    keepdims=True)
    masked = jnp.where(e_iota == top1i, _NEG_INF, noisy)
    top2v = jnp.max(masked, axis=0, keepdims=True)
    top2i = jnp.min(jnp.where(masked == top2v, e_iota, E), axis=0,
                    keepdims=True)

    ed = jnp.exp(top2v - top1v)
    denom = 1.0 + ed
    g1 = 1.0 / denom
    g2 = ed / denom

    sel1 = e_iota == top1i
    sel2 = e_iota == top2i
    sel = jnp.logical_or(sel1, sel2)
    self32 = sel.astype(jnp.float32)

    r_iota = lax.broadcasted_iota(jnp.int32, (F, F), 0)
    c_iota = lax.broadcasted_iota(jnp.int32, (F, F), 1)
    ut = (r_iota <= c_iota).astype(jnp.float32)
    ranks = jnp.dot(self32, ut, preferred_element_type=jnp.float32)
    posf = jnp.where(sel, ranks - 1.0, float(_BIG))
    pos = posf.astype(jnp.int32)

    gate = jnp.where(sel1, g1, jnp.where(sel2, g2, 0.0))
    gate = jnp.where(pos < cap, gate, 0.0)

    pos_ref[...] = pos
    gate_ref[...] = gate

    counts = ranks[:, F - 1:F]
    e_col = lax.broadcasted_iota(jnp.int32, (E, 1), 0)
    estar = jnp.min(jnp.where(counts < cap, e_col, E), axis=0,
                    keepdims=True)
    counts_ref[...] = counts.astype(jnp.int32)

    o_drop = estar * cap + (cap - 1)
    pos1 = jnp.sum(jnp.where(sel1, posf, 0.0), axis=0, keepdims=True)
    pos2 = jnp.sum(jnp.where(sel2, posf, 0.0), axis=0, keepdims=True)
    o1 = jnp.where(pos1 < cap, top1i * cap + pos1.astype(jnp.int32), o_drop)
    o2 = jnp.where(pos2 < cap, top2i * cap + pos2.astype(jnp.int32), o_drop)
    meta_ref[0:1, :] = o1
    meta_ref[1:2, :] = o2

    tok_iota = lax.broadcasted_iota(jnp.int32, (1, F), 1).astype(jnp.float32)
    slot_iota = lax.broadcasted_iota(jnp.int32, (cap, F), 0)
    for e in range(E):
        p_e = (pos[e:e + 1, :] == slot_iota).astype(jnp.float32)
        tok_col = jnp.sum(p_e * tok_iota, axis=1, keepdims=True)
        g_col = jnp.sum(p_e * gate[e:e + 1, :], axis=1, keepdims=True)
        tok_ref[pl.ds(e * cap, cap), :] = tok_col.astype(jnp.int32)
        gslot_ref[pl.ds(e * cap, cap), :] = g_col


def _ffn_body(counts_ref, gslot_ref, xg_ref, w1_ref, b1_ref,
              w2_ref, b2_ref, out_ref, y_scr, *, cap, n_ffb):
    e = pl.program_id(0)
    f = pl.program_id(1)
    count = jnp.minimum(counts_ref[e], cap)
    blocks = []
    start = 0
    while start < cap:
        size = min(_RB, cap - start)
        blocks.append((start, size))
        start += size

    for start, size in blocks:
        @pl.when(start < count)
        def _compute(start=start, size=size):
            rows = pl.ds(start, size)
            h = jnp.dot(xg_ref[rows, :], w1_ref[0],
                        preferred_element_type=jnp.float32)
            h = h + b1_ref[0]
            h = h * 0.5 * (1.0 + lax.erf(h * 0.7071067811865476))
            yb = jnp.dot(h, w2_ref[0], preferred_element_type=jnp.float32)

            @pl.when(f == 0)
            def _init():
                y_scr[rows, :] = yb

            @pl.when(f != 0)
            def _acc():
                y_scr[rows, :] = y_scr[rows, :] + yb

    @pl.when(f == n_ffb - 1)
    def _emit():
        for start, size in blocks:
            rows = pl.ds(start, size)

            @pl.when(start < count)
            def _write(rows=rows):
                out_ref[rows, :] = ((y_scr[rows, :] + b2_ref[0])
                                    * gslot_ref[rows, :])

            @pl.when(start >= count)
            def _zero(rows=rows):
                out_ref[rows, :] = jnp.zeros_like(out_ref[rows, :])


def _gather_body(flat_hbm, tok_hbm, out_hbm, idx_v, rows_v0, rows_v1,
                 sem0, sem1, *, rows_per_w, chunk):
    wid = lax.axis_index("s") * _NC + lax.axis_index("c")
    base = wid * rows_per_w
    pltpu.sync_copy(tok_hbm.at[pl.ds(base, rows_per_w)], idx_v)
    bufs = (rows_v0, rows_v1)
    sems = (sem0, sem1)
    n = rows_per_w // chunk
    cps = [None] * n
    for c in range(min(2, n)):
        cps[c] = pltpu.async_copy(
            flat_hbm.at[idx_v.at[pl.ds(c * chunk, chunk)]], bufs[c % 2],
            sems[c % 2])
    for c in range(n):
        cps[c].wait()
        pltpu.sync_copy(bufs[c % 2],
                        out_hbm.at[pl.ds(base + c * chunk, chunk)])
        if c + 2 < n:
            cps[c + 2] = pltpu.async_copy(
                flat_hbm.at[idx_v.at[pl.ds((c + 2) * chunk, chunk)]],
                bufs[c % 2], sems[c % 2])


def _combine_body(ybuf_hbm, o1_hbm, o2_hbm, out_hbm,
                  idx1_v, idx2_v, r1_v, r2_v, sem1, sem2,
                  *, toks_per_w, chunk, d):
    wid = lax.axis_index("s") * _NC + lax.axis_index("c")
    base = wid * toks_per_w
    pltpu.sync_copy(o1_hbm.at[pl.ds(base, toks_per_w)], idx1_v)
    pltpu.sync_copy(o2_hbm.at[pl.ds(base, toks_per_w)], idx2_v)
    n_chunks = toks_per_w // chunk
    n_lane = d // 16
    for c in range(n_chunks):
        cp1 = pltpu.async_copy(
            ybuf_hbm.at[idx1_v.at[pl.ds(c * chunk, chunk)]], r1_v, sem1)
        cp2 = pltpu.async_copy(
            ybuf_hbm.at[idx2_v.at[pl.ds(c * chunk, chunk)]], r2_v, sem2)
        cp1.wait()
        cp2.wait()

        def _row(r, _):
            def _col(j, _):
                for u in range(4):
                    cols = pl.ds(j * 64 + u * 16, 16)
                    r1_v[r, cols] = r1_v[r, cols] + r2_v[r, cols]
                return _
            return lax.fori_loop(0, n_lane // 4, _col, _)

        lax.fori_loop(0, chunk, _row, None)
        pltpu.sync_copy(r1_v, out_hbm.at[pl.ds(base + c * chunk, chunk)])


def kernel(x_BLD, W_route, b_route, W_noise, b_noise, W1, b1, W2, b2):
    Bs, Ls, Ds = x_BLD.shape
    F = Bs * Ls
    E = W_route.shape[1]
    FF = W1.shape[2]
    cap = int(F * 2 / E * 1.25)
    flat = x_BLD.reshape(F, Ds)
    noise_mat = jax.random.normal(jax.random.key(1234), (F, E),
                                  dtype=jnp.float32)

    pos, gate, tok, gslot, meta, counts = pl.pallas_call(
        functools.partial(_router_body, cap=cap),
        out_shape=(
            jax.ShapeDtypeStruct((E, F), jnp.int32),
            jax.ShapeDtypeStruct((E, F), jnp.float32),
            jax.ShapeDtypeStruct((E * cap, 1), jnp.int32),
            jax.ShapeDtypeStruct((E * cap, 1), jnp.float32),
            jax.ShapeDtypeStruct((2, F), jnp.int32),
            jax.ShapeDtypeStruct((E, 1), jnp.int32),
        ),
    )(flat, W_route, b_route.reshape(E, 1), W_noise, b_noise.reshape(E, 1),
      noise_mat.T)

    nw = _NC * _NS
    rows_per_w = (E * cap) // nw
    mesh = plsc.VectorSubcoreMesh(core_axis_name="c", subcore_axis_name="s",
                                  num_cores=_NC, num_subcores=_NS)
    g_chunk = 40
    xg = pl.kernel(
        functools.partial(_gather_body, rows_per_w=rows_per_w,
                          chunk=g_chunk),
        out_type=jax.ShapeDtypeStruct((E * cap, Ds), jnp.float32),
        mesh=mesh,
        scratch_types=[
            pltpu.VMEM((rows_per_w,), jnp.int32),
            pltpu.VMEM((g_chunk, Ds), jnp.float32),
            pltpu.VMEM((g_chunk, Ds), jnp.float32),
            pltpu.SemaphoreType.DMA,
            pltpu.SemaphoreType.DMA,
        ],
    )(flat, tok.reshape(E * cap))

    n_ffb = 2
    ffb = FF // n_ffb
    ybuf = pl.pallas_call(
        functools.partial(_ffn_body, cap=cap, n_ffb=n_ffb),
        grid=(E, n_ffb),
        in_specs=[
            pl.BlockSpec(memory_space=pltpu.SMEM),
            pl.BlockSpec((cap, 1), lambda e, f: (e, 0)),
            pl.BlockSpec((cap, Ds), lambda e, f: (e, 0)),
            pl.BlockSpec((1, Ds, ffb), lambda e, f: (e, 0, f)),
            pl.BlockSpec((1, 1, ffb), lambda e, f: (e, 0, f)),
            pl.BlockSpec((1, ffb, Ds), lambda e, f: (e, f, 0)),
            pl.BlockSpec((1, 1, Ds), lambda e, f: (e, 0, 0)),
        ],
        out_specs=pl.BlockSpec((cap, Ds), lambda e, f: (e, 0)),
        out_shape=jax.ShapeDtypeStruct((E * cap, Ds), jnp.float32),
        scratch_shapes=[
            pltpu.VMEM((cap, Ds), jnp.float32),
        ],
    )(counts.reshape(E), gslot, xg, W1, b1.reshape(E, 1, FF), W2,
      b2.reshape(E, 1, Ds))

    toks_per_w = F // nw
    c_chunk = 32
    out = pl.kernel(
        functools.partial(_combine_body, toks_per_w=toks_per_w,
                          chunk=c_chunk, d=Ds),
        out_type=jax.ShapeDtypeStruct((F, Ds), jnp.float32),
        mesh=mesh,
        scratch_types=[
            pltpu.VMEM((toks_per_w,), jnp.int32),
            pltpu.VMEM((toks_per_w,), jnp.int32),
            pltpu.VMEM((c_chunk, Ds), jnp.float32),
            pltpu.VMEM((c_chunk, Ds), jnp.float32),
            pltpu.SemaphoreType.DMA,
            pltpu.SemaphoreType.DMA,
        ],
    )(ybuf, meta[0], meta[1])

    return out.reshape(Bs, Ls, Ds)


# speedup vs baseline: 1.5613x; 1.2882x over previous
import functools

import jax
import jax.numpy as jnp
from jax import lax
from jax.experimental import pallas as pl
from jax.experimental.pallas import tpu as pltpu
from jax.experimental.pallas import tpu_sc as plsc

_NEG_INF = float("-inf")
_BIG = 1 << 20
_NC = 2
_NS = 16
_RB = 256


def _router_body(x_ref, wr_ref, br_ref, wn_ref, bn_ref, nz_ref,
                 pos_ref, gate_ref, gslot_ref, meta_ref,
                 counts_ref, *, cap):
    x = x_ref[...]
    E = wr_ref.shape[1]
    F = x.shape[0]
    dn = (((0,), (1,)), ((), ()))
    logits = lax.dot_general(wr_ref[...], x, dn,
                             preferred_element_type=jnp.float32)
    logits = logits + br_ref[...]
    nlogits = lax.dot_general(wn_ref[...], x, dn,
                              preferred_element_type=jnp.float32)
    nlogits = nlogits + bn_ref[...]
    sp = jnp.log1p(jnp.exp(-jnp.abs(nlogits))) + jnp.maximum(nlogits, 0.0)
    noisy = logits + nz_ref[...] * sp

    e_iota = lax.broadcasted_iota(jnp.int32, noisy.shape, 0)
    top1v = jnp.max(noisy, axis=0, keepdims=True)
    top1i = jnp.min(jnp.where(noisy == top1v, e_iota, E), axis=0,
                    keepdims=True)
    masked = jnp.where(e_iota == top1i, _NEG_INF, noisy)
    top2v = jnp.max(masked, axis=0, keepdims=True)
    top2i = jnp.min(jnp.where(masked == top2v, e_iota, E), axis=0,
                    keepdims=True)

    ed = jnp.exp(top2v - top1v)
    denom = 1.0 + ed
    g1 = 1.0 / denom
    g2 = ed / denom

    sel1 = e_iota == top1i
    sel2 = e_iota == top2i
    sel = jnp.logical_or(sel1, sel2)
    self32 = sel.astype(jnp.float32)

    r_iota = lax.broadcasted_iota(jnp.int32, (F, F), 0)
    c_iota = lax.broadcasted_iota(jnp.int32, (F, F), 1)
    ut = (r_iota <= c_iota).astype(jnp.float32)
    ranks = jnp.dot(self32, ut, preferred_element_type=jnp.float32)
    posf = jnp.where(sel, ranks - 1.0, float(_BIG))
    pos = posf.astype(jnp.int32)

    gate = jnp.where(sel1, g1, jnp.where(sel2, g2, 0.0))
    gate = jnp.where(pos < cap, gate, 0.0)

    pos_ref[...] = pos
    gate_ref[...] = gate

    counts = ranks[:, F - 1:F]
    e_col = lax.broadcasted_iota(jnp.int32, (E, 1), 0)
    estar = jnp.min(jnp.where(counts < cap, e_col, E), axis=0,
                    keepdims=True)
    counts_ref[...] = counts.astype(jnp.int32)

    o_drop = estar * cap + (cap - 1)
    pos1 = jnp.sum(jnp.where(sel1, posf, 0.0), axis=0, keepdims=True)
    pos2 = jnp.sum(jnp.where(sel2, posf, 0.0), axis=0, keepdims=True)
    o1 = jnp.where(pos1 < cap, top1i * cap + pos1.astype(jnp.int32), o_drop)
    o2 = jnp.where(pos2 < cap, top2i * cap + pos2.astype(jnp.int32), o_drop)
    meta_ref[0:1, :] = o1
    meta_ref[1:2, :] = o2

    slot_iota = lax.broadcasted_iota(jnp.int32, (cap, F), 0)
    for e in range(E):
        p_e = (pos[e:e + 1, :] == slot_iota).astype(jnp.float32)
        g_col = jnp.sum(p_e * gate[e:e + 1, :], axis=1, keepdims=True)
        gslot_ref[pl.ds(e * cap, cap), :] = g_col


def _ffn_body(counts_ref, pos_ref, gslot_ref, x_ref, w1_ref, b1_ref,
              w2_ref, b2_ref, out_ref, xg_scr, y_scr, *, cap, n_ffb):
    e = pl.program_id(0)
    f = pl.program_id(1)
    count = jnp.minimum(counts_ref[e], cap)
    blocks = []
    start = 0
    while start < cap:
        size = min(_RB, cap - start)
        blocks.append((start, size))
        start += size

    for start, size in blocks:
        @pl.when(jnp.logical_and(start < count, f == 0))
        def _gather(start=start, size=size):
            pos_row = pos_ref[0]
            slot_iota = start + lax.broadcasted_iota(
                jnp.int32, (size, pos_row.shape[1]), 0)
            p_blk = (pos_row == slot_iota).astype(jnp.float32)
            xg_scr[pl.ds(start, size), :] = jnp.dot(
                p_blk, x_ref[...], preferred_element_type=jnp.float32)

    for start, size in blocks:
        @pl.when(start < count)
        def _compute(start=start, size=size):
            rows = pl.ds(start, size)
            h = jnp.dot(xg_scr[rows, :], w1_ref[0],
                        preferred_element_type=jnp.float32)
            h = h + b1_ref[0]
            h = h * 0.5 * (1.0 + lax.erf(h * 0.7071067811865476))
            yb = jnp.dot(h, w2_ref[0], preferred_element_type=jnp.float32)

            @pl.when(f == 0)
            def _init():
                y_scr[rows, :] = yb

            @pl.when(f != 0)
            def _acc():
                y_scr[rows, :] = y_scr[rows, :] + yb

    @pl.when(f == n_ffb - 1)
    def _emit():
        for start, size in blocks:
            rows = pl.ds(start, size)

            @pl.when(start < count)
            def _write(rows=rows):
                out_ref[rows, :] = ((y_scr[rows, :] + b2_ref[0])
                                    * gslot_ref[rows, :])

            @pl.when(start >= count)
            def _zero(rows=rows):
                out_ref[rows, :] = jnp.zeros_like(out_ref[rows, :])


def _combine_body(ybuf_hbm, o1_hbm, o2_hbm, out_hbm,
                  idx1_v, idx2_v, r1_v, r2_v, sem1, sem2,
                  *, toks_per_w, chunk, d):
    wid = lax.axis_index("s") * _NC + lax.axis_index("c")
    base = wid * toks_per_w
    pltpu.sync_copy(o1_hbm.at[pl.ds(base, toks_per_w)], idx1_v)
    pltpu.sync_copy(o2_hbm.at[pl.ds(base, toks_per_w)], idx2_v)
    n_chunks = toks_per_w // chunk
    n_lane = d // 16
    for c in range(n_chunks):
        cp1 = pltpu.async_copy(
            ybuf_hbm.at[idx1_v.at[pl.ds(c * chunk, chunk)]], r1_v, sem1)
        cp2 = pltpu.async_copy(
            ybuf_hbm.at[idx2_v.at[pl.ds(c * chunk, chunk)]], r2_v, sem2)
        cp1.wait()
        cp2.wait()

        def _row(r, _):
            def _col(j, _):
                for u in range(4):
                    cols = pl.ds(j * 64 + u * 16, 16)
                    r1_v[r, cols] = r1_v[r, cols] + r2_v[r, cols]
                return _
            return lax.fori_loop(0, n_lane // 4, _col, _)

        lax.fori_loop(0, chunk, _row, None)
        pltpu.sync_copy(r1_v, out_hbm.at[pl.ds(base + c * chunk, chunk)])


def kernel(x_BLD, W_route, b_route, W_noise, b_noise, W1, b1, W2, b2):
    Bs, Ls, Ds = x_BLD.shape
    F = Bs * Ls
    E = W_route.shape[1]
    FF = W1.shape[2]
    cap = int(F * 2 / E * 1.25)
    flat = x_BLD.reshape(F, Ds)
    noise_mat = jax.random.normal(jax.random.key(1234), (F, E),
                                  dtype=jnp.float32)

    pos, gate, gslot, meta, counts = pl.pallas_call(
        functools.partial(_router_body, cap=cap),
        out_shape=(
            jax.ShapeDtypeStruct((E, F), jnp.int32),
            jax.ShapeDtypeStruct((E, F), jnp.float32),
            jax.ShapeDtypeStruct((E * cap, 1), jnp.float32),
            jax.ShapeDtypeStruct((2, F), jnp.int32),
            jax.ShapeDtypeStruct((E, 1), jnp.int32),
        ),
    )(flat, W_route, b_route.reshape(E, 1), W_noise, b_noise.reshape(E, 1),
      noise_mat.T)

    nw = _NC * _NS
    mesh = plsc.VectorSubcoreMesh(core_axis_name="c", subcore_axis_name="s",
                                  num_cores=_NC, num_subcores=_NS)

    n_ffb = 2
    ffb = FF // n_ffb
    ybuf = pl.pallas_call(
        functools.partial(_ffn_body, cap=cap, n_ffb=n_ffb),
        grid=(E, n_ffb),
        in_specs=[
            pl.BlockSpec(memory_space=pltpu.SMEM),
            pl.BlockSpec((1, 1, F), lambda e, f: (e, 0, 0)),
            pl.BlockSpec((cap, 1), lambda e, f: (e, 0)),
            pl.BlockSpec((F, Ds), lambda e, f: (0, 0)),
            pl.BlockSpec((1, Ds, ffb), lambda e, f: (e, 0, f)),
            pl.BlockSpec((1, 1, ffb), lambda e, f: (e, 0, f)),
            pl.BlockSpec((1, ffb, Ds), lambda e, f: (e, f, 0)),
            pl.BlockSpec((1, 1, Ds), lambda e, f: (e, 0, 0)),
        ],
        out_specs=pl.BlockSpec((cap, Ds), lambda e, f: (e, 0)),
        out_shape=jax.ShapeDtypeStruct((E * cap, Ds), jnp.float32),
        scratch_shapes=[
            pltpu.VMEM((cap, Ds), jnp.float32),
            pltpu.VMEM((cap, Ds), jnp.float32),
        ],
    )(counts.reshape(E), pos.reshape(E, 1, F), gslot, flat, W1,
      b1.reshape(E, 1, FF), W2, b2.reshape(E, 1, Ds))

    toks_per_w = F // nw
    c_chunk = 32
    out = pl.kernel(
        functools.partial(_combine_body, toks_per_w=toks_per_w,
                          chunk=c_chunk, d=Ds),
        out_type=jax.ShapeDtypeStruct((F, Ds), jnp.float32),
        mesh=mesh,
        scratch_types=[
            pltpu.VMEM((toks_per_w,), jnp.int32),
            pltpu.VMEM((toks_per_w,), jnp.int32),
            pltpu.VMEM((c_chunk, Ds), jnp.float32),
            pltpu.VMEM((c_chunk, Ds), jnp.float32),
            pltpu.SemaphoreType.DMA,
            pltpu.SemaphoreType.DMA,
        ],
    )(ybuf, meta[0], meta[1])

    return out.reshape(Bs, Ls, Ds)


# speedup vs baseline: 1.5921x; 1.0197x over previous
import functools

import jax
import jax.numpy as jnp
from jax import lax
from jax.experimental import pallas as pl
from jax.experimental.pallas import tpu as pltpu
from jax.experimental.pallas import tpu_sc as plsc

_NEG_INF = float("-inf")
_BIG = 1 << 20
_NC = 2
_NS = 16
_RB = 256


def _router_body(x_ref, wr_ref, br_ref, wn_ref, bn_ref, nz_ref,
                 pos_ref, gate_ref, gslot_ref, meta_ref,
                 counts_ref, *, cap):
    x = x_ref[...]
    E = wr_ref.shape[1]
    F = x.shape[0]
    dn = (((0,), (1,)), ((), ()))
    logits = lax.dot_general(wr_ref[...], x, dn,
                             preferred_element_type=jnp.float32)
    logits = logits + br_ref[...]
    nlogits = lax.dot_general(wn_ref[...], x, dn,
                              preferred_element_type=jnp.float32)
    nlogits = nlogits + bn_ref[...]
    sp = jnp.log1p(jnp.exp(-jnp.abs(nlogits))) + jnp.maximum(nlogits, 0.0)
    noisy = logits + nz_ref[...] * sp

    e_iota = lax.broadcasted_iota(jnp.int32, noisy.shape, 0)
    top1v = jnp.max(noisy, axis=0, keepdims=True)
    top1i = jnp.min(jnp.where(noisy == top1v, e_iota, E), axis=0,
                    keepdims=True)
    masked = jnp.where(e_iota == top1i, _NEG_INF, noisy)
    top2v = jnp.max(masked, axis=0, keepdims=True)
    top2i = jnp.min(jnp.where(masked == top2v, e_iota, E), axis=0,
                    keepdims=True)

    ed = jnp.exp(top2v - top1v)
    denom = 1.0 + ed
    g1 = 1.0 / denom
    g2 = ed / denom

    sel1 = e_iota == top1i
    sel2 = e_iota == top2i
    sel = jnp.logical_or(sel1, sel2)
    self32 = sel.astype(jnp.float32)

    r_iota = lax.broadcasted_iota(jnp.int32, (F, F), 0)
    c_iota = lax.broadcasted_iota(jnp.int32, (F, F), 1)
    ut = (r_iota <= c_iota).astype(jnp.float32)
    ranks = jnp.dot(self32, ut, preferred_element_type=jnp.float32)
    posf = jnp.where(sel, ranks - 1.0, float(_BIG))
    pos = posf.astype(jnp.int32)

    gate = jnp.where(sel1, g1, jnp.where(sel2, g2, 0.0))
    gate = jnp.where(pos < cap, gate, 0.0)

    pos_ref[...] = pos
    gate_ref[...] = gate

    counts = ranks[:, F - 1:F]
    e_col = lax.broadcasted_iota(jnp.int32, (E, 1), 0)
    estar = jnp.min(jnp.where(counts < cap, e_col, E), axis=0,
                    keepdims=True)
    counts_ref[...] = counts.astype(jnp.int32)

    o_drop = estar * cap + (cap - 1)
    pos1 = jnp.sum(jnp.where(sel1, posf, 0.0), axis=0, keepdims=True)
    pos2 = jnp.sum(jnp.where(sel2, posf, 0.0), axis=0, keepdims=True)
    o1 = jnp.where(pos1 < cap, top1i * cap + pos1.astype(jnp.int32), o_drop)
    o2 = jnp.where(pos2 < cap, top2i * cap + pos2.astype(jnp.int32), o_drop)
    meta_ref[0:1, :] = o1
    meta_ref[1:2, :] = o2

    slot_iota = lax.broadcasted_iota(jnp.int32, (cap, F), 0)
    for e in range(E):
        p_e = (pos[e:e + 1, :] == slot_iota).astype(jnp.float32)
        g_col = jnp.sum(p_e * gate[e:e + 1, :], axis=1, keepdims=True)
        gslot_ref[pl.ds(e * cap, cap), :] = g_col


def _ffn_body(counts_ref, pos_ref, gslot_ref, x_ref, w1_ref, b1_ref,
              w2_ref, b2_ref, out_ref, xg_scr, y_scr, *, cap, n_ffb):
    e = pl.program_id(0)
    f = pl.program_id(1)
    count = jnp.minimum(counts_ref[e], cap)
    blocks = []
    start = 0
    while start < cap:
        size = min(_RB, cap - start)
        blocks.append((start, size))
        start += size

    for start, size in blocks:
        @pl.when(jnp.logical_and(start < count, f == 0))
        def _gather(start=start, size=size):
            pos_row = pos_ref[0]
            slot_iota = start + lax.broadcasted_iota(
                jnp.int32, (size, pos_row.shape[1]), 0)
            p_blk = (pos_row == slot_iota).astype(jnp.float32)
            xg_scr[pl.ds(start, size), :] = jnp.dot(
                p_blk, x_ref[...], preferred_element_type=jnp.float32)

    for start, size in blocks:
        @pl.when(start < count)
        def _compute(start=start, size=size):
            rows = pl.ds(start, size)
            h = jnp.dot(xg_scr[rows, :], w1_ref[0],
                        preferred_element_type=jnp.float32)
            h = h + b1_ref[0]
            h = h * 0.5 * (1.0 + lax.erf(h * 0.7071067811865476))
            yb = jnp.dot(h, w2_ref[0], preferred_element_type=jnp.float32)

            @pl.when(f == 0)
            def _init():
                y_scr[rows, :] = yb

            @pl.when(f != 0)
            def _acc():
                y_scr[rows, :] = y_scr[rows, :] + yb

    @pl.when(f == n_ffb - 1)
    def _emit():
        for start, size in blocks:
            rows = pl.ds(start, size)

            @pl.when(start < count)
            def _write(rows=rows):
                out_ref[rows, :] = ((y_scr[rows, :] + b2_ref[0])
                                    * gslot_ref[rows, :])

            @pl.when(start >= count)
            def _zero(rows=rows):
                out_ref[rows, :] = jnp.zeros_like(out_ref[rows, :])


def _combine_body(ybuf_hbm, o1_hbm, o2_hbm, out_hbm,
                  idx1_v, idx2_v, r1a_v, r2a_v, r1b_v, r2b_v,
                  sem1a, sem2a, sem1b, sem2b,
                  *, toks_per_w, chunk, d):
    wid = lax.axis_index("s") * _NC + lax.axis_index("c")
    base = wid * toks_per_w
    pltpu.sync_copy(o1_hbm.at[pl.ds(base, toks_per_w)], idx1_v)
    pltpu.sync_copy(o2_hbm.at[pl.ds(base, toks_per_w)], idx2_v)
    n_chunks = toks_per_w // chunk
    n_lane = d // 16
    r1 = (r1a_v, r1b_v)
    r2 = (r2a_v, r2b_v)
    s1 = (sem1a, sem1b)
    s2 = (sem2a, sem2b)

    def _start(c):
        return (pltpu.async_copy(
                    ybuf_hbm.at[idx1_v.at[pl.ds(c * chunk, chunk)]],
                    r1[c % 2], s1[c % 2]),
                pltpu.async_copy(
                    ybuf_hbm.at[idx2_v.at[pl.ds(c * chunk, chunk)]],
                    r2[c % 2], s2[c % 2]))

    cps = [None] * n_chunks
    cps[0] = _start(0)
    for c in range(n_chunks):
        cps[c][0].wait()
        cps[c][1].wait()
        if c + 1 < n_chunks:
            cps[c + 1] = _start(c + 1)
        r1c, r2c = r1[c % 2], r2[c % 2]

        def _row(r, _):
            def _col(j, _):
                for u in range(4):
                    cols = pl.ds(j * 64 + u * 16, 16)
                    r1c[r, cols] = r1c[r, cols] + r2c[r, cols]
                return _
            return lax.fori_loop(0, n_lane // 4, _col, _)

        lax.fori_loop(0, chunk, _row, None)
        pltpu.sync_copy(r1c, out_hbm.at[pl.ds(base + c * chunk, chunk)])


def kernel(x_BLD, W_route, b_route, W_noise, b_noise, W1, b1, W2, b2):
    Bs, Ls, Ds = x_BLD.shape
    F = Bs * Ls
    E = W_route.shape[1]
    FF = W1.shape[2]
    cap = int(F * 2 / E * 1.25)
    flat = x_BLD.reshape(F, Ds)
    noise_mat = jax.random.normal(jax.random.key(1234), (F, E),
                                  dtype=jnp.float32)

    pos, gate, gslot, meta, counts = pl.pallas_call(
        functools.partial(_router_body, cap=cap),
        out_shape=(
            jax.ShapeDtypeStruct((E, F), jnp.int32),
            jax.ShapeDtypeStruct((E, F), jnp.float32),
            jax.ShapeDtypeStruct((E * cap, 1), jnp.float32),
            jax.ShapeDtypeStruct((2, F), jnp.int32),
            jax.ShapeDtypeStruct((E, 1), jnp.int32),
        ),
    )(flat, W_route, b_route.reshape(E, 1), W_noise, b_noise.reshape(E, 1),
      noise_mat.T)

    nw = _NC * _NS
    mesh = plsc.VectorSubcoreMesh(core_axis_name="c", subcore_axis_name="s",
                                  num_cores=_NC, num_subcores=_NS)

    n_ffb = 2
    ffb = FF // n_ffb
    ybuf = pl.pallas_call(
        functools.partial(_ffn_body, cap=cap, n_ffb=n_ffb),
        grid=(E, n_ffb),
        in_specs=[
            pl.BlockSpec(memory_space=pltpu.SMEM),
            pl.BlockSpec((1, 1, F), lambda e, f: (e, 0, 0)),
            pl.BlockSpec((cap, 1), lambda e, f: (e, 0)),
            pl.BlockSpec((F, Ds), lambda e, f: (0, 0)),
            pl.BlockSpec((1, Ds, ffb), lambda e, f: (e, 0, f)),
            pl.BlockSpec((1, 1, ffb), lambda e, f: (e, 0, f)),
            pl.BlockSpec((1, ffb, Ds), lambda e, f: (e, f, 0)),
            pl.BlockSpec((1, 1, Ds), lambda e, f: (e, 0, 0)),
        ],
        out_specs=pl.BlockSpec((cap, Ds), lambda e, f: (e, 0)),
        out_shape=jax.ShapeDtypeStruct((E * cap, Ds), jnp.float32),
        scratch_shapes=[
            pltpu.VMEM((cap, Ds), jnp.float32),
            pltpu.VMEM((cap, Ds), jnp.float32),
        ],
    )(counts.reshape(E), pos.reshape(E, 1, F), gslot, flat, W1,
      b1.reshape(E, 1, FF), W2, b2.reshape(E, 1, Ds))

    toks_per_w = F // nw
    c_chunk = 16
    out = pl.kernel(
        functools.partial(_combine_body, toks_per_w=toks_per_w,
                          chunk=c_chunk, d=Ds),
        out_type=jax.ShapeDtypeStruct((F, Ds), jnp.float32),
        mesh=mesh,
        scratch_types=[
            pltpu.VMEM((toks_per_w,), jnp.int32),
            pltpu.VMEM((toks_per_w,), jnp.int32),
            pltpu.VMEM((c_chunk, Ds), jnp.float32),
            pltpu.VMEM((c_chunk, Ds), jnp.float32),
            pltpu.VMEM((c_chunk, Ds), jnp.float32),
            pltpu.VMEM((c_chunk, Ds), jnp.float32),
            pltpu.SemaphoreType.DMA,
            pltpu.SemaphoreType.DMA,
            pltpu.SemaphoreType.DMA,
            pltpu.SemaphoreType.DMA,
        ],
    )(ybuf, meta[0], meta[1])

    return out.reshape(Bs, Ls, Ds)


# speedup vs baseline: 1.6215x; 1.0185x over previous
import functools

import jax
import jax.numpy as jnp
from jax import lax
from jax.experimental import pallas as pl
from jax.experimental.pallas import tpu as pltpu
from jax.experimental.pallas import tpu_sc as plsc

_NEG_INF = float("-inf")
_BIG = 1 << 20
_NC = 2
_NS = 16
_RB = 256


def _router_body(x_ref, wr_ref, br_ref, wn_ref, bn_ref, nz_ref,
                 pos_ref, gate_ref, meta_ref, counts_ref, *, cap):
    x = x_ref[...]
    E = wr_ref.shape[1]
    F = x.shape[0]
    dn = (((0,), (1,)), ((), ()))
    logits = lax.dot_general(wr_ref[...], x, dn,
                             preferred_element_type=jnp.float32)
    logits = logits + br_ref[...]
    nlogits = lax.dot_general(wn_ref[...], x, dn,
                              preferred_element_type=jnp.float32)
    nlogits = nlogits + bn_ref[...]
    sp = jnp.log1p(jnp.exp(-jnp.abs(nlogits))) + jnp.maximum(nlogits, 0.0)
    noisy = logits + nz_ref[...] * sp

    e_iota = lax.broadcasted_iota(jnp.int32, noisy.shape, 0)
    top1v = jnp.max(noisy, axis=0, keepdims=True)
    top1i = jnp.min(jnp.where(noisy == top1v, e_iota, E), axis=0,
                    keepdims=True)
    masked = jnp.where(e_iota == top1i, _NEG_INF, noisy)
    top2v = jnp.max(masked, axis=0, keepdims=True)
    top2i = jnp.min(jnp.where(masked == top2v, e_iota, E), axis=0,
                    keepdims=True)

    ed = jnp.exp(top2v - top1v)
    denom = 1.0 + ed
    g1 = 1.0 / denom
    g2 = ed / denom

    sel1 = e_iota == top1i
    sel2 = e_iota == top2i
    sel = jnp.logical_or(sel1, sel2)
    self32 = sel.astype(jnp.float32)

    r_iota = lax.broadcasted_iota(jnp.int32, (F, F), 0)
    c_iota = lax.broadcasted_iota(jnp.int32, (F, F), 1)
    ut = (r_iota <= c_iota).astype(jnp.float32)
    ranks = jnp.dot(self32, ut, preferred_element_type=jnp.float32)
    posf = jnp.where(sel, ranks - 1.0, float(_BIG))
    pos = posf.astype(jnp.int32)

    gate = jnp.where(sel1, g1, jnp.where(sel2, g2, 0.0))
    gate = jnp.where(pos < cap, gate, 0.0)

    pos_ref[...] = pos
    gate_ref[...] = gate

    counts = ranks[:, F - 1:F]
    e_col = lax.broadcasted_iota(jnp.int32, (E, 1), 0)
    estar = jnp.min(jnp.where(counts < cap, e_col, E), axis=0,
                    keepdims=True)
    counts_ref[...] = counts.astype(jnp.int32)

    o_drop = estar * cap + (cap - 1)
    pos1 = jnp.sum(jnp.where(sel1, posf, 0.0), axis=0, keepdims=True)
    pos2 = jnp.sum(jnp.where(sel2, posf, 0.0), axis=0, keepdims=True)
    o1 = jnp.where(pos1 < cap, top1i * cap + pos1.astype(jnp.int32), o_drop)
    o2 = jnp.where(pos2 < cap, top2i * cap + pos2.astype(jnp.int32), o_drop)
    meta_ref[0:1, :] = o1
    meta_ref[1:2, :] = o2


def _ffn_body(counts_ref, pos_ref, gate_ref, x_ref, w1_ref, b1_ref,
              w2_ref, b2_ref, out_ref, xg_scr, y_scr, gs_scr,
              *, cap, n_ffb):
    e = pl.program_id(0)
    f = pl.program_id(1)
    count = jnp.minimum(counts_ref[e], cap)
    blocks = []
    start = 0
    while start < cap:
        size = min(_RB, cap - start)
        blocks.append((start, size))
        start += size

    for start, size in blocks:
        @pl.when(jnp.logical_and(start < count, f == 0))
        def _gather(start=start, size=size):
            pos_row = pos_ref[0]
            slot_iota = start + lax.broadcasted_iota(
                jnp.int32, (size, pos_row.shape[1]), 0)
            p_blk = (pos_row == slot_iota).astype(jnp.float32)
            xg_scr[pl.ds(start, size), :] = jnp.dot(
                p_blk, x_ref[...], preferred_element_type=jnp.float32)
            gs_scr[pl.ds(start, size), :] = jnp.sum(
                p_blk * gate_ref[0], axis=1, keepdims=True)

    for start, size in blocks:
        @pl.when(start < count)
        def _compute(start=start, size=size):
            rows = pl.ds(start, size)
            h = jnp.dot(xg_scr[rows, :], w1_ref[0],
                        preferred_element_type=jnp.float32)
            h = h + b1_ref[0]
            h = h * 0.5 * (1.0 + lax.erf(h * 0.7071067811865476))
            yb = jnp.dot(h, w2_ref[0], preferred_element_type=jnp.float32)

            @pl.when(f == 0)
            def _init():
                y_scr[rows, :] = yb

            @pl.when(f != 0)
            def _acc():
                y_scr[rows, :] = y_scr[rows, :] + yb

    @pl.when(f == n_ffb - 1)
    def _emit():
        for start, size in blocks:
            rows = pl.ds(start, size)

            @pl.when(start < count)
            def _write(rows=rows):
                out_ref[rows, :] = ((y_scr[rows, :] + b2_ref[0])
                                    * gs_scr[rows, :])

            @pl.when(start >= count)
            def _zero(rows=rows):
                out_ref[rows, :] = jnp.zeros_like(out_ref[rows, :])


def _combine_body(ybuf_hbm, o1_hbm, o2_hbm, out_hbm,
                  idx1_v, idx2_v, r1a_v, r2a_v, r1b_v, r2b_v,
                  sem1a, sem2a, sem1b, sem2b,
                  *, toks_per_w, chunk, d):
    wid = lax.axis_index("s") * _NC + lax.axis_index("c")
    base = wid * toks_per_w
    pltpu.sync_copy(o1_hbm.at[pl.ds(base, toks_per_w)], idx1_v)
    pltpu.sync_copy(o2_hbm.at[pl.ds(base, toks_per_w)], idx2_v)
    n_chunks = toks_per_w // chunk
    n_lane = d // 16
    r1 = (r1a_v, r1b_v)
    r2 = (r2a_v, r2b_v)
    s1 = (sem1a, sem1b)
    s2 = (sem2a, sem2b)

    def _start(c):
        return (pltpu.async_copy(
                    ybuf_hbm.at[idx1_v.at[pl.ds(c * chunk, chunk)]],
                    r1[c % 2], s1[c % 2]),
                pltpu.async_copy(
                    ybuf_hbm.at[idx2_v.at[pl.ds(c * chunk, chunk)]],
                    r2[c % 2], s2[c % 2]))

    cps = [None] * n_chunks
    cps[0] = _start(0)
    for c in range(n_chunks):
        cps[c][0].wait()
        cps[c][1].wait()
        if c + 1 < n_chunks:
            cps[c + 1] = _start(c + 1)
        r1c, r2c = r1[c % 2], r2[c % 2]

        def _row(r, _):
            def _col(j, _):
                for u in range(4):
                    cols = pl.ds(j * 64 + u * 16, 16)
                    r1c[r, cols] = r1c[r, cols] + r2c[r, cols]
                return _
            return lax.fori_loop(0, n_lane // 4, _col, _)

        lax.fori_loop(0, chunk, _row, None)
        pltpu.sync_copy(r1c, out_hbm.at[pl.ds(base + c * chunk, chunk)])


def kernel(x_BLD, W_route, b_route, W_noise, b_noise, W1, b1, W2, b2):
    Bs, Ls, Ds = x_BLD.shape
    F = Bs * Ls
    E = W_route.shape[1]
    FF = W1.shape[2]
    cap = int(F * 2 / E * 1.25)
    flat = x_BLD.reshape(F, Ds)
    noise_mat = jax.random.normal(jax.random.key(1234), (F, E),
                                  dtype=jnp.float32)

    pos, gate, meta, counts = pl.pallas_call(
        functools.partial(_router_body, cap=cap),
        out_shape=(
            jax.ShapeDtypeStruct((E, F), jnp.int32),
            jax.ShapeDtypeStruct((E, F), jnp.float32),
            jax.ShapeDtypeStruct((2, F), jnp.int32),
            jax.ShapeDtypeStruct((E, 1), jnp.int32),
        ),
    )(flat, W_route, b_route.reshape(E, 1), W_noise, b_noise.reshape(E, 1),
      noise_mat.T)

    nw = _NC * _NS
    mesh = plsc.VectorSubcoreMesh(core_axis_name="c", subcore_axis_name="s",
                                  num_cores=_NC, num_subcores=_NS)

    n_ffb = 2
    ffb = FF // n_ffb
    ybuf = pl.pallas_call(
        functools.partial(_ffn_body, cap=cap, n_ffb=n_ffb),
        grid=(E, n_ffb),
        in_specs=[
            pl.BlockSpec(memory_space=pltpu.SMEM),
            pl.BlockSpec((1, 1, F), lambda e, f: (e, 0, 0)),
            pl.BlockSpec((1, 1, F), lambda e, f: (e, 0, 0)),
            pl.BlockSpec((F, Ds), lambda e, f: (0, 0)),
            pl.BlockSpec((1, Ds, ffb), lambda e, f: (e, 0, f)),
            pl.BlockSpec((1, 1, ffb), lambda e, f: (e, 0, f)),
            pl.BlockSpec((1, ffb, Ds), lambda e, f: (e, f, 0)),
            pl.BlockSpec((1, 1, Ds), lambda e, f: (e, 0, 0)),
        ],
        out_specs=pl.BlockSpec((cap, Ds), lambda e, f: (e, 0)),
        out_shape=jax.ShapeDtypeStruct((E * cap, Ds), jnp.float32),
        scratch_shapes=[
            pltpu.VMEM((cap, Ds), jnp.float32),
            pltpu.VMEM((cap, Ds), jnp.float32),
            pltpu.VMEM((cap, 1), jnp.float32),
        ],
    )(counts.reshape(E), pos.reshape(E, 1, F), gate.reshape(E, 1, F), flat,
      W1, b1.reshape(E, 1, FF), W2, b2.reshape(E, 1, Ds))

    toks_per_w = F // nw
    c_chunk = 16
    out = pl.kernel(
        functools.partial(_combine_body, toks_per_w=toks_per_w,
                          chunk=c_chunk, d=Ds),
        out_type=jax.ShapeDtypeStruct((F, Ds), jnp.float32),
        mesh=mesh,
        scratch_types=[
            pltpu.VMEM((toks_per_w,), jnp.int32),
            pltpu.VMEM((toks_per_w,), jnp.int32),
            pltpu.VMEM((c_chunk, Ds), jnp.float32),
            pltpu.VMEM((c_chunk, Ds), jnp.float32),
            pltpu.VMEM((c_chunk, Ds), jnp.float32),
            pltpu.VMEM((c_chunk, Ds), jnp.float32),
            pltpu.SemaphoreType.DMA,
            pltpu.SemaphoreType.DMA,
            pltpu.SemaphoreType.DMA,
            pltpu.SemaphoreType.DMA,
        ],
    )(ybuf, meta[0], meta[1])

    return out.reshape(Bs, Ls, Ds)
